# baseline (speedup 1.0000x reference)
import functools

import jax
import jax.numpy as jnp
from jax import lax
from jax.experimental import pallas as pl
from jax.experimental.pallas import tpu as pltpu
from jax.experimental.pallas import tpu_sc as plsc

_NG_LIST = (2, 3)
_NUM_HEADS = 4
_EV = 1024
_BASES = (31, 37, 41, 43, 47, 53, 59, 61)

_NC = 2
_NS = 16
_L = 16
_NW = _NC * _NS


def _ids_body(x_ref, out_ref):
    x = x_ref[...]
    b, s = x.shape
    z = jnp.zeros((b, 1), jnp.int32)
    x1 = jnp.concatenate([z, x[:, :-1]], axis=1)
    x2 = jnp.concatenate([z, z, x[:, :-2]], axis=1)
    pos = lax.broadcasted_iota(jnp.int32, (b, s), 1)
    i = 0
    for n in _NG_LIST:
        for k in range(_NUM_HEADS):
            base = _BASES[k % len(_BASES)]
            if n == 2:
                hv = x1 * base + x * (base + 1)
            else:
                hv = x2 * base + x1 * (base + 1) + x * (base + 2)
            hv = hv % (_EV - 1) + 1
            hv = jnp.where(pos >= n - 1, hv, 0)
            out_ref[i] = hv + i * _EV
            i += 1


def _fold_body(e_ref, w_ref, m_ref):
    m_ref[0] = lax.dot_general(
        e_ref[0].astype(jnp.bfloat16), w_ref[...].astype(jnp.bfloat16),
        dimension_numbers=(((1,), (1,)), ((), ())),
        preferred_element_type=jnp.float32)


def _sc_gather_sum_body(ntok, hid, tck, m_hbm, idx_hbm, out_hbm,
                        idx_v, acc_v, buf0_v, buf1_v, sg0, sg1, sacc, sout):
    tpw = ntok // _NW
    nchunk = tpw // tck
    jpr = hid // _L
    jshift = jpr.bit_length() - 1
    assert jpr == 1 << jshift
    nslice = tck * jpr

    wid = lax.axis_index("s") * _NC + lax.axis_index("c")
    base = wid * tpw
    pltpu.sync_copy(idx_hbm.at[:, pl.ds(base, tpw)], idx_v)

    bufs = (buf0_v, buf1_v)
    gsems = (sg0, sg1)

    def gidx(ti, ci):
        return idx_v.at[ti, pl.ds(ci * tck, tck)]

    cp_acc = pltpu.make_async_copy(m_hbm.at[gidx(0, 0)], acc_v, sacc)
    cp_acc.start()
    g = 0
    pend = pltpu.make_async_copy(m_hbm.at[gidx(1, 0)], bufs[0], gsems[0])
    pend.start()
    out_cp = None

    for ci in range(nchunk):
        cp_acc.wait()
        for ti in range(1, 8):
            cur, src = pend, bufs[g % 2]
            nti, nci = (ti + 1, ci) if ti < 7 else (1, ci + 1)
            if nci < nchunk:
                g += 1
                pend = pltpu.make_async_copy(
                    m_hbm.at[gidx(nti, nci)], bufs[g % 2], gsems[g % 2])
                pend.start()
            cur.wait()

            @plsc.parallel_loop(0, nslice, 1, unroll=16)
            def _(q, src=src):
                t = lax.shift_right_logical(q, jshift)
                c = jnp.bitwise_and(q, jpr - 1) * _L
                plsc.addupdate(acc_v.at[t, pl.ds(c, _L)], src[t, pl.ds(c, _L)])

        out_cp = pltpu.make_async_copy(
            acc_v, out_hbm.at[pl.ds(base + ci * tck, tck)], sout)
        out_cp.start()
        if ci + 1 < nchunk:
            out_cp.wait()
            cp_acc = pltpu.make_async_copy(m_hbm.at[gidx(0, ci + 1)], acc_v, sacc)
            cp_acc.start()
    out_cp.wait()


def _alpha_body(h_ref, m_ref, gh_ref, gm_ref, bd_ref, bg_ref, a_ref):
    h = h_ref[...]
    m = m_ref[...]
    gh = gh_ref[...]
    gm = gm_ref[...]
    const = jnp.sum(bd_ref[...] * gm) + bg_ref[0, 0]
    s = jnp.sum(h * gh + m * gm, axis=1, keepdims=True) + const
    a_ref[...] = jax.nn.sigmoid(s)


def _fuse_conv_body(h_ref, m_ref, a_ref, bd_ref, cw_ref, o_ref):
    h = h_ref[0]
    m = m_ref[0] + bd_ref[...]
    a = a_ref[0]
    f = h + a * m
    s, cb = f.shape
    z = jnp.zeros((1, cb), jnp.float32)
    f1 = jnp.concatenate([z, f[:-1, :]], axis=0)
    f2 = jnp.concatenate([z, z, f[:-2, :]], axis=0)
    f3 = jnp.concatenate([z, z, z, f[:-3, :]], axis=0)
    w = cw_ref[...]
    o_ref[0] = f * w[3:4] + f1 * w[2:3] + f2 * w[1:2] + f3 * w[0:1]


def kernel(hidden_states, input_ids, emb_tables, W_down, b_down, W_gate,
           b_gate, conv_w):
    B, S, H = hidden_states.shape
    ntab = emb_tables.shape[0]
    ntok = B * S

    ids8 = pl.pallas_call(
        _ids_body,
        out_shape=jax.ShapeDtypeStruct((ntab, B, S), jnp.int32),
    )(input_ids)
    idx_flat = ids8.reshape(ntab, ntok)

    ot = 512
    M = pl.pallas_call(
        _fold_body,
        grid=(ntab, H // ot),
        in_specs=[
            pl.BlockSpec((1, _EV, H), lambda i, j: (i, 0, 0)),
            pl.BlockSpec((ot, H), lambda i, j: (j, i)),
        ],
        out_specs=pl.BlockSpec((1, _EV, ot), lambda i, j: (i, 0, j)),
        out_shape=jax.ShapeDtypeStruct((ntab, _EV, H), jnp.float32),
    )(emb_tables, W_down)
    M2 = M.reshape(ntab * _EV, H)

    tpw = ntok // _NW
    tck = 16
    mesh = plsc.VectorSubcoreMesh(core_axis_name="c", subcore_axis_name="s")
    sc_fn = pl.kernel(
        functools.partial(_sc_gather_sum_body, ntok, H, tck),
        out_type=jax.ShapeDtypeStruct((ntok, H), jnp.float32),
        mesh=mesh,
        scratch_types=[
            pltpu.VMEM((ntab, tpw), jnp.int32),
            pltpu.VMEM((tck, H), jnp.float32),
            pltpu.VMEM((tck, H), jnp.float32),
            pltpu.VMEM((tck, H), jnp.float32),
            pltpu.SemaphoreType.DMA,
            pltpu.SemaphoreType.DMA,
            pltpu.SemaphoreType.DMA,
            pltpu.SemaphoreType.DMA,
        ],
    )
    mem_sum = sc_fn(M2, idx_flat)

    gh = W_gate[:, :H]
    gm = W_gate[:, H:]
    bd2 = b_down.reshape(1, H)
    bg2 = b_gate.reshape(1, 1)
    h_flat = hidden_states.reshape(ntok, H)
    tb = 256
    alpha = pl.pallas_call(
        _alpha_body,
        grid=(ntok // tb,),
        in_specs=[
            pl.BlockSpec((tb, H), lambda i: (i, 0)),
            pl.BlockSpec((tb, H), lambda i: (i, 0)),
            pl.BlockSpec((1, H), lambda i: (0, 0)),
            pl.BlockSpec((1, H), lambda i: (0, 0)),
            pl.BlockSpec((1, H), lambda i: (0, 0)),
            pl.BlockSpec((1, 1), lambda i: (0, 0)),
        ],
        out_specs=pl.BlockSpec((tb, 1), lambda i: (i, 0)),
        out_shape=jax.ShapeDtypeStruct((ntok, 1), jnp.float32),
    )(h_flat, mem_sum, gh, gm, bd2, bg2)

    cb = 512
    mem3 = mem_sum.reshape(B, S, H)
    alpha3 = alpha.reshape(B, S, 1)
    cwT = conv_w[:, 0, :].T
    out = pl.pallas_call(
        _fuse_conv_body,
        grid=(B, H // cb),
        in_specs=[
            pl.BlockSpec((1, S, cb), lambda b, j: (b, 0, j)),
            pl.BlockSpec((1, S, cb), lambda b, j: (b, 0, j)),
            pl.BlockSpec((1, S, 1), lambda b, j: (b, 0, 0)),
            pl.BlockSpec((1, cb), lambda b, j: (0, j)),
            pl.BlockSpec((4, cb), lambda b, j: (0, j)),
        ],
        out_specs=pl.BlockSpec((1, S, cb), lambda b, j: (b, 0, j)),
        out_shape=jax.ShapeDtypeStruct((B, S, H), jnp.float32),
    )(hidden_states, mem3, alpha3, bd2, cwT)
    return out


# speedup vs baseline: 1.3547x; 1.3547x over previous
import functools

import jax
import jax.numpy as jnp
from jax import lax
from jax.experimental import pallas as pl
from jax.experimental.pallas import tpu as pltpu
from jax.experimental.pallas import tpu_sc as plsc

_NG_LIST = (2, 3)
_NUM_HEADS = 4
_EV = 1024
_BASES = (31, 37, 41, 43, 47, 53, 59, 61)

_NC = 2
_NS = 16
_L = 16
_NW = _NC * _NS


def _ids_body(x_ref, out_ref):
    x = x_ref[...]
    b, s = x.shape
    z = jnp.zeros((b, 1), jnp.int32)
    x1 = jnp.concatenate([z, x[:, :-1]], axis=1)
    x2 = jnp.concatenate([z, z, x[:, :-2]], axis=1)
    pos = lax.broadcasted_iota(jnp.int32, (b, s), 1)
    i = 0
    for n in _NG_LIST:
        for k in range(_NUM_HEADS):
            base = _BASES[k % len(_BASES)]
            if n == 2:
                hv = x1 * base + x * (base + 1)
            else:
                hv = x2 * base + x1 * (base + 1) + x * (base + 2)
            hv = hv % (_EV - 1) + 1
            hv = jnp.where(pos >= n - 1, hv, 0)
            out_ref[i] = hv + i * _EV
            i += 1


_MASK_HI = -65536


def _rne_bf16_bits(x):
    u = lax.bitcast_convert_type(x, jnp.int32)
    return u + 0x7FFF + jnp.bitwise_and(lax.shift_right_logical(u, 16), 1)


def _fold_body(e_ref, w_ref, m_ref):
    x = lax.dot_general(
        e_ref[0].astype(jnp.bfloat16), w_ref[...].astype(jnp.bfloat16),
        dimension_numbers=(((1,), (1,)), ((), ())),
        preferred_element_type=jnp.float32)
    hw = x.shape[1] // 2
    r_lo = _rne_bf16_bits(x[:, :hw])
    r_hi = _rne_bf16_bits(x[:, hw:])
    m_ref[0] = jnp.bitwise_or(jnp.bitwise_and(r_hi, _MASK_HI),
                              lax.shift_right_logical(r_lo, 16))


def _sc_gather_sum_body(ntok, hid, tck, m_hbm, idx_hbm, out_hbm,
                        idx_v, acc_v, buf0_v, buf1_v, sg0, sg1, sacc, sout):
    hw = hid // 2
    tpw = ntok // _NW
    nchunk = tpw // tck
    jpr = hw // _L
    jshift = jpr.bit_length() - 1
    assert jpr == 1 << jshift
    nslice = tck * jpr

    wid = lax.axis_index("s") * _NC + lax.axis_index("c")
    base = wid * tpw
    pltpu.sync_copy(idx_hbm.at[:, pl.ds(base, tpw)], idx_v)

    bufs = (buf0_v, buf1_v)
    gsems = (sg0, sg1)

    def gidx(ti, ci):
        return idx_v.at[ti, pl.ds(ci * tck, tck)]

    cp_acc = pltpu.make_async_copy(m_hbm.at[gidx(0, 0)], acc_v, sacc)
    cp_acc.start()
    g = 0
    pend = pltpu.make_async_copy(m_hbm.at[gidx(1, 0)], bufs[0], gsems[0])
    pend.start()
    out_cp = None

    for ci in range(nchunk):
        cp_acc.wait()
        for ti in range(1, 8):
            cur, src = pend, bufs[g % 2]
            nti, nci = (ti + 1, ci) if ti < 7 else (1, ci + 1)
            if nci < nchunk:
                g += 1
                pend = pltpu.make_async_copy(
                    m_hbm.at[gidx(nti, nci)], bufs[g % 2], gsems[g % 2])
                pend.start()
            cur.wait()
            accb = acc_v.bitcast(jnp.bfloat16)
            srcb = src.bitcast(jnp.bfloat16)

            @plsc.parallel_loop(0, nslice, 1, unroll=8)
            def _(q, accb=accb, srcb=srcb):
                t2 = lax.shift_right_logical(q, jshift) * 2
                cw = jnp.bitwise_and(q, jpr - 1) * _L
                sl = (pl.ds(t2, 2), pl.ds(cw, _L))
                aref = accb.at[sl]
                sref = srcb.at[sl]
                aref.set(aref.get() + sref.get())

        out_cp = pltpu.make_async_copy(
            acc_v, out_hbm.at[pl.ds(base + ci * tck, tck)], sout)
        out_cp.start()
        if ci + 1 < nchunk:
            out_cp.wait()
            cp_acc = pltpu.make_async_copy(m_hbm.at[gidx(0, ci + 1)], acc_v, sacc)
            cp_acc.start()
    out_cp.wait()


def _unpack_lo(x):
    return lax.bitcast_convert_type(lax.shift_left(x, 16), jnp.float32)


def _unpack_hi(x):
    return lax.bitcast_convert_type(jnp.bitwise_and(x, _MASK_HI), jnp.float32)


def _alpha_body(h_ref, m_ref, gh_ref, gmp_ref, bdp_ref, bg_ref, a_ref):
    h = h_ref[...]
    x = m_ref[...]
    mfull = jnp.concatenate([_unpack_lo(x), _unpack_hi(x)], axis=1)
    gh = gh_ref[...]
    gmp = gmp_ref[...]
    const = jnp.sum(bdp_ref[...] * gmp) + bg_ref[0, 0]
    s = (jnp.sum(h * gh, axis=1, keepdims=True)
         + jnp.sum(mfull * gmp, axis=1, keepdims=True) + const)
    a_ref[...] = jax.nn.sigmoid(s)


def _fuse_conv_body(h_ref, m_ref, a_ref, bd_ref, cw_ref, o_ref):
    h = h_ref[0]
    x = m_ref[0]
    bd = bd_ref[...]
    a = a_ref[0]
    m = jnp.concatenate([_unpack_lo(x), _unpack_hi(x)], axis=1) + bd
    f = h + a * m
    s, cb = f.shape
    z = jnp.zeros((1, cb), jnp.float32)
    f1 = jnp.concatenate([z, f[:-1, :]], axis=0)
    f2 = jnp.concatenate([z, z, f[:-2, :]], axis=0)
    f3 = jnp.concatenate([z, z, z, f[:-3, :]], axis=0)
    w = cw_ref[...]
    o_ref[0] = f * w[3:4] + f1 * w[2:3] + f2 * w[1:2] + f3 * w[0:1]


def kernel(hidden_states, input_ids, emb_tables, W_down, b_down, W_gate,
           b_gate, conv_w):
    B, S, H = hidden_states.shape
    ntab = emb_tables.shape[0]
    ntok = B * S

    ids8 = pl.pallas_call(
        _ids_body,
        out_shape=jax.ShapeDtypeStruct((ntab, B, S), jnp.int32),
    )(input_ids)
    idx_flat = ids8.reshape(ntab, ntok)

    hw = H // 2
    ts = 512
    M = pl.pallas_call(
        _fold_body,
        grid=(ntab, H // ts),
        in_specs=[
            pl.BlockSpec((1, _EV, H), lambda i, j: (i, 0, 0)),
            pl.BlockSpec((ts, H), lambda i, j: (j, i)),
        ],
        out_specs=pl.BlockSpec((1, _EV, ts // 2), lambda i, j: (i, 0, j)),
        out_shape=jax.ShapeDtypeStruct((ntab, _EV, hw), jnp.int32),
    )(emb_tables, W_down)
    M2 = M.reshape(ntab * _EV, hw)

    tpw = ntok // _NW
    tck = 32
    mesh = plsc.VectorSubcoreMesh(core_axis_name="c", subcore_axis_name="s")
    sc_fn = pl.kernel(
        functools.partial(_sc_gather_sum_body, ntok, H, tck),
        out_type=jax.ShapeDtypeStruct((ntok, hw), jnp.int32),
        mesh=mesh,
        scratch_types=[
            pltpu.VMEM((ntab, tpw), jnp.int32),
            pltpu.VMEM((tck, hw), jnp.int32),
            pltpu.VMEM((tck, hw), jnp.int32),
            pltpu.VMEM((tck, hw), jnp.int32),
            pltpu.SemaphoreType.DMA,
            pltpu.SemaphoreType.DMA,
            pltpu.SemaphoreType.DMA,
            pltpu.SemaphoreType.DMA,
        ],
    )
    mem_packed = sc_fn(M2, idx_flat)

    gh = W_gate[:, :H]
    gm = W_gate[:, H:]
    bd2 = b_down.reshape(1, H)

    def _perm(v):
        q = ts // 2
        lo = [v[:, j * ts:j * ts + q] for j in range(H // ts)]
        hi = [v[:, j * ts + q:(j + 1) * ts] for j in range(H // ts)]
        return jnp.concatenate(lo + hi, axis=1)

    gmp = _perm(gm)
    bdp = _perm(bd2)
    bg2 = b_gate.reshape(1, 1)
    h_flat = hidden_states.reshape(ntok, H)
    tb = 256
    alpha = pl.pallas_call(
        _alpha_body,
        grid=(ntok // tb,),
        in_specs=[
            pl.BlockSpec((tb, H), lambda i: (i, 0)),
            pl.BlockSpec((tb, hw), lambda i: (i, 0)),
            pl.BlockSpec((1, H), lambda i: (0, 0)),
            pl.BlockSpec((1, H), lambda i: (0, 0)),
            pl.BlockSpec((1, H), lambda i: (0, 0)),
            pl.BlockSpec((1, 1), lambda i: (0, 0)),
        ],
        out_specs=pl.BlockSpec((tb, 1), lambda i: (i, 0)),
        out_shape=jax.ShapeDtypeStruct((ntok, 1), jnp.float32),
    )(h_flat, mem_packed, gh, gmp, bdp, bg2)

    cb = ts
    mem3 = mem_packed.reshape(B, S, hw)
    alpha3 = alpha.reshape(B, S, 1)
    cwT = conv_w[:, 0, :].T
    out = pl.pallas_call(
        _fuse_conv_body,
        grid=(B, H // cb),
        in_specs=[
            pl.BlockSpec((1, S, cb), lambda b, j: (b, 0, j)),
            pl.BlockSpec((1, S, cb // 2), lambda b, j: (b, 0, j)),
            pl.BlockSpec((1, S, 1), lambda b, j: (b, 0, 0)),
            pl.BlockSpec((1, cb), lambda b, j: (0, j)),
            pl.BlockSpec((4, cb), lambda b, j: (0, j)),
        ],
        out_specs=pl.BlockSpec((1, S, cb), lambda b, j: (b, 0, j)),
        out_shape=jax.ShapeDtypeStruct((B, S, H), jnp.float32),
    )(hidden_states, mem3, alpha3, bd2, cwT)
    return out


# speedup vs baseline: 1.4126x; 1.0428x over previous
import functools

import jax
import jax.numpy as jnp
from jax import lax
from jax.experimental import pallas as pl
from jax.experimental.pallas import tpu as pltpu
from jax.experimental.pallas import tpu_sc as plsc

_NG_LIST = (2, 3)
_NUM_HEADS = 4
_EV = 1024
_BASES = (31, 37, 41, 43, 47, 53, 59, 61)

_NC = 2
_NS = 16
_L = 16
_NW = _NC * _NS


def _ids_body(gsize, x_ref, out_ref):
    x = x_ref[...]
    b, s = x.shape
    z = jnp.zeros((b, 1), jnp.int32)
    x1 = jnp.concatenate([z, x[:, :-1]], axis=1)
    x2 = jnp.concatenate([z, z, x[:, :-2]], axis=1)
    pos = lax.broadcasted_iota(jnp.int32, (b, s), 1)
    i = 0
    for n in _NG_LIST:
        for k in range(_NUM_HEADS):
            base = _BASES[k % len(_BASES)]
            if n == 2:
                hv = x1 * base + x * (base + 1)
            else:
                hv = x2 * base + x1 * (base + 1) + x * (base + 2)
            hv = hv % (_EV - 1) + 1
            hv = jnp.where(pos >= n - 1, hv, 0)
            out_ref[i] = hv + (i % gsize) * _EV
            i += 1


_MASK_HI = -65536


def _rne_bf16_bits(x):
    u = lax.bitcast_convert_type(x, jnp.int32)
    return u + 0x7FFF + jnp.bitwise_and(lax.shift_right_logical(u, 16), 1)


def _fold_body(e_ref, w_ref, m_ref):
    x = lax.dot_general(
        e_ref[0].astype(jnp.bfloat16), w_ref[...].astype(jnp.bfloat16),
        dimension_numbers=(((1,), (1,)), ((), ())),
        preferred_element_type=jnp.float32)
    hw = x.shape[1] // 2
    r_lo = _rne_bf16_bits(x[:, :hw])
    r_hi = _rne_bf16_bits(x[:, hw:])
    m_ref[0] = jnp.bitwise_or(jnp.bitwise_and(r_hi, _MASK_HI),
                              lax.shift_right_logical(r_lo, 16))


def _sc_gather_sum_body(ntok, hid, tck, nt, m_hbm, idx_hbm, out_hbm,
                        idx_v, acc_v, buf0_v, buf1_v, sg0, sg1, sacc, sout):
    hw = hid // 2
    tpw = ntok // _NW
    nchunk = tpw // tck
    jpr = hw // _L
    jshift = jpr.bit_length() - 1
    assert jpr == 1 << jshift
    nslice = tck * jpr

    wid = lax.axis_index("s") * _NC + lax.axis_index("c")
    base = wid * tpw
    pltpu.sync_copy(idx_hbm.at[:, pl.ds(base, tpw)], idx_v)

    bufs = (buf0_v, buf1_v)
    gsems = (sg0, sg1)

    def gidx(ti, ci):
        return idx_v.at[ti, pl.ds(ci * tck, tck)]

    cp_acc = pltpu.make_async_copy(m_hbm.at[gidx(0, 0)], acc_v, sacc)
    cp_acc.start()
    g = 0
    pend = pltpu.make_async_copy(m_hbm.at[gidx(1, 0)], bufs[0], gsems[0])
    pend.start()
    out_cp = None

    for ci in range(nchunk):
        cp_acc.wait()
        for ti in range(1, nt):
            cur, src = pend, bufs[g % 2]
            nti, nci = (ti + 1, ci) if ti < nt - 1 else (1, ci + 1)
            if nci < nchunk:
                g += 1
                pend = pltpu.make_async_copy(
                    m_hbm.at[gidx(nti, nci)], bufs[g % 2], gsems[g % 2])
                pend.start()
            cur.wait()
            accb = acc_v.bitcast(jnp.bfloat16)
            srcb = src.bitcast(jnp.bfloat16)

            @plsc.parallel_loop(0, nslice, 1, unroll=8)
            def _(q, accb=accb, srcb=srcb):
                t2 = lax.shift_right_logical(q, jshift) * 2
                cw = jnp.bitwise_and(q, jpr - 1) * _L
                sl = (pl.ds(t2, 2), pl.ds(cw, _L))
                aref = accb.at[sl]
                sref = srcb.at[sl]
                aref.set(aref.get() + sref.get())

        out_cp = pltpu.make_async_copy(
            acc_v, out_hbm.at[pl.ds(base + ci * tck, tck)], sout)
        out_cp.start()
        if ci + 1 < nchunk:
            out_cp.wait()
            cp_acc = pltpu.make_async_copy(m_hbm.at[gidx(0, ci + 1)], acc_v, sacc)
            cp_acc.start()
    out_cp.wait()


def _unpack_lo(x):
    return lax.bitcast_convert_type(lax.shift_left(x, 16), jnp.float32)


def _unpack_hi(x):
    return lax.bitcast_convert_type(jnp.bitwise_and(x, _MASK_HI), jnp.float32)


def _alpha_body(h_ref, ma_ref, mb_ref, gh_ref, gmp_ref, bdp_ref, bg_ref,
                a_ref):
    h = h_ref[...]
    xa = ma_ref[...]
    xb = mb_ref[...]
    mfull = jnp.concatenate([_unpack_lo(xa) + _unpack_lo(xb),
                             _unpack_hi(xa) + _unpack_hi(xb)], axis=1)
    gh = gh_ref[...]
    gmp = gmp_ref[...]
    const = jnp.sum(bdp_ref[...] * gmp) + bg_ref[0, 0]
    s = (jnp.sum(h * gh, axis=1, keepdims=True)
         + jnp.sum(mfull * gmp, axis=1, keepdims=True) + const)
    a_ref[...] = jax.nn.sigmoid(s)


def _fuse_conv_body(h_ref, ma_ref, mb_ref, a_ref, bd_ref, cw_ref, o_ref):
    h = h_ref[0]
    xa = ma_ref[0]
    xb = mb_ref[0]
    bd = bd_ref[...]
    a = a_ref[0]
    m = jnp.concatenate([_unpack_lo(xa) + _unpack_lo(xb),
                         _unpack_hi(xa) + _unpack_hi(xb)], axis=1) + bd
    f = h + a * m
    s, cb = f.shape
    z = jnp.zeros((1, cb), jnp.float32)
    f1 = jnp.concatenate([z, f[:-1, :]], axis=0)
    f2 = jnp.concatenate([z, z, f[:-2, :]], axis=0)
    f3 = jnp.concatenate([z, z, z, f[:-3, :]], axis=0)
    w = cw_ref[...]
    o_ref[0] = f * w[3:4] + f1 * w[2:3] + f2 * w[1:2] + f3 * w[0:1]


def kernel(hidden_states, input_ids, emb_tables, W_down, b_down, W_gate,
           b_gate, conv_w):
    B, S, H = hidden_states.shape
    ntab = emb_tables.shape[0]
    ntok = B * S

    ngrp = 2
    gsize = ntab // ngrp
    ids8 = pl.pallas_call(
        functools.partial(_ids_body, gsize),
        out_shape=jax.ShapeDtypeStruct((ntab, B, S), jnp.int32),
    )(input_ids)
    idx_flat = ids8.reshape(ntab, ntok)

    hw = H // 2
    ts = 512
    tpw = ntok // _NW
    tck = 32
    mesh = plsc.VectorSubcoreMesh(core_axis_name="c", subcore_axis_name="s")
    sc_fn = pl.kernel(
        functools.partial(_sc_gather_sum_body, ntok, H, tck, gsize),
        out_type=jax.ShapeDtypeStruct((ntok, hw), jnp.int32),
        mesh=mesh,
        scratch_types=[
            pltpu.VMEM((gsize, tpw), jnp.int32),
            pltpu.VMEM((tck, hw), jnp.int32),
            pltpu.VMEM((tck, hw), jnp.int32),
            pltpu.VMEM((tck, hw), jnp.int32),
            pltpu.SemaphoreType.DMA,
            pltpu.SemaphoreType.DMA,
            pltpu.SemaphoreType.DMA,
            pltpu.SemaphoreType.DMA,
        ],
    )

    mems = []
    for gi in range(ngrp):
        toff = gi * gsize
        Mg = pl.pallas_call(
            _fold_body,
            grid=(gsize, H // ts),
            in_specs=[
                pl.BlockSpec((1, _EV, H),
                             lambda i, j, toff=toff: (i + toff, 0, 0)),
                pl.BlockSpec((ts, H),
                             lambda i, j, toff=toff: (j, i + toff)),
            ],
            out_specs=pl.BlockSpec((1, _EV, ts // 2), lambda i, j: (i, 0, j)),
            out_shape=jax.ShapeDtypeStruct((gsize, _EV, hw), jnp.int32),
        )(emb_tables, W_down)
        mems.append(sc_fn(Mg.reshape(gsize * _EV, hw),
                          idx_flat[toff:toff + gsize]))
    mem_a, mem_b = mems

    gh = W_gate[:, :H]
    gm = W_gate[:, H:]
    bd2 = b_down.reshape(1, H)

    def _perm(v):
        q = ts // 2
        lo = [v[:, j * ts:j * ts + q] for j in range(H // ts)]
        hi = [v[:, j * ts + q:(j + 1) * ts] for j in range(H // ts)]
        return jnp.concatenate(lo + hi, axis=1)

    gmp = _perm(gm)
    bdp = _perm(bd2)
    bg2 = b_gate.reshape(1, 1)
    h_flat = hidden_states.reshape(ntok, H)
    tb = 256
    alpha = pl.pallas_call(
        _alpha_body,
        grid=(ntok // tb,),
        in_specs=[
            pl.BlockSpec((tb, H), lambda i: (i, 0)),
            pl.BlockSpec((tb, hw), lambda i: (i, 0)),
            pl.BlockSpec((tb, hw), lambda i: (i, 0)),
            pl.BlockSpec((1, H), lambda i: (0, 0)),
            pl.BlockSpec((1, H), lambda i: (0, 0)),
            pl.BlockSpec((1, H), lambda i: (0, 0)),
            pl.BlockSpec((1, 1), lambda i: (0, 0)),
        ],
        out_specs=pl.BlockSpec((tb, 1), lambda i: (i, 0)),
        out_shape=jax.ShapeDtypeStruct((ntok, 1), jnp.float32),
    )(h_flat, mem_a, mem_b, gh, gmp, bdp, bg2)

    cb = ts
    mem3a = mem_a.reshape(B, S, hw)
    mem3b = mem_b.reshape(B, S, hw)
    alpha3 = alpha.reshape(B, S, 1)
    cwT = conv_w[:, 0, :].T
    out = pl.pallas_call(
        _fuse_conv_body,
        grid=(B, H // cb),
        in_specs=[
            pl.BlockSpec((1, S, cb), lambda b, j: (b, 0, j)),
            pl.BlockSpec((1, S, cb // 2), lambda b, j: (b, 0, j)),
            pl.BlockSpec((1, S, cb // 2), lambda b, j: (b, 0, j)),
            pl.BlockSpec((1, S, 1), lambda b, j: (b, 0, 0)),
            pl.BlockSpec((1, cb), lambda b, j: (0, j)),
            pl.BlockSpec((4, cb), lambda b, j: (0, j)),
        ],
        out_specs=pl.BlockSpec((1, S, cb), lambda b, j: (b, 0, j)),
        out_shape=jax.ShapeDtypeStruct((B, S, H), jnp.float32),
    )(hidden_states, mem3a, mem3b, alpha3, bd2, cwT)
    return out
